# Initial kernel scaffold; baseline (speedup 1.0000x reference)
import jax
import jax.numpy as jnp
from jax.experimental import pallas as pl


def kernel(input_spikes, max_timesteps, T0, W0, T1, W1, T2, W2):
    raise NotImplementedError("write your pallas kernel here")


# speedup vs baseline: 107.0618x; 107.0618x over previous
import functools
import math

import numpy as np

import jax
import jax.numpy as jnp
from jax import lax
from jax.experimental import pallas as pl
from jax.experimental.pallas import tpu as pltpu
from jax.experimental.pallas import tpu_sc as plsc

N = 10000
NOUT = 128
FAN = 64
NP = 10240
NW = 32
RPT = NP // NW
NROW = RPT * FAN // 128
TAU = 20.0
THR = 1.0
DECAY = float(np.float32(math.exp(-1.0 / 20.0)))

_MESH = plsc.VectorSubcoreMesh(core_axis_name="c", subcore_axis_name="s")


def _wid():
    return lax.axis_index("c") * 16 + lax.axis_index("s")


def _zero_slice(zb_v, acc_sh, sid, span):
    zeros = jnp.zeros((16,), jnp.float32)
    for i in range(zb_v.shape[0] // 16):
        zb_v[pl.ds(i * 16, 16)] = zeros
    pltpu.sync_copy(zb_v.at[pl.ds(0, span)], acc_sh.at[pl.ds(sid * span, span)])


def _threshold_spikes(p_hbm, base, p0_v, p1_v, sp_v):
    pltpu.sync_copy(p_hbm.at[pl.ds(base, RPT)], p0_v)
    pltpu.sync_copy(p_hbm.at[pl.ds(NP + base, RPT)], p1_v)
    for i in range(RPT // 16):
        s = pl.ds(i * 16, 16)
        p = (p0_v[s] + p1_v[s]) * DECAY
        sp_v[s] = jnp.where(p >= THR, 1.0, 0.0).astype(jnp.float32)


def _scatter_stage(t_hbm, w_hbm, out_hbm, idx_v, w_v, sp_v, zb_v, acc_sh, load_spikes):
    cid = lax.axis_index("c")
    sid = lax.axis_index("s")
    wid = cid * 16 + sid
    _zero_slice(zb_v, acc_sh, sid, NP // 16)
    pltpu.sync_copy(t_hbm.at[wid], idx_v)
    pltpu.sync_copy(w_hbm.at[wid], w_v)
    load_spikes(wid)
    plsc.subcore_barrier()

    def mul_body(r, carry):
        for c8 in range(8):
            off = pl.multiple_of(lax.rem(r * 128 + c8 * 16, RPT), 16)
            s = pl.ds(c8 * 16, 16)
            w_v[r, s] = w_v[r, s] * sp_v[pl.ds(off, 16)]
        return carry

    lax.fori_loop(0, NROW, mul_body, 0)

    def sc_body(r, carry):
        pltpu.sync_copy(w_v.at[r], acc_sh.at[idx_v.at[r]], add=True)
        return carry

    lax.fori_loop(0, NROW, sc_body, 0)
    plsc.subcore_barrier()
    span = NP // 16
    pltpu.sync_copy(acc_sh.at[pl.ds(sid * span, span)],
                    out_hbm.at[pl.ds(cid * NP + sid * span, span)])


_SCATTER_SCRATCH = [
    pltpu.VMEM((NROW, 128), jnp.int32),
    pltpu.VMEM((NROW, 128), jnp.float32),
    pltpu.VMEM((RPT,), jnp.float32),
    pltpu.VMEM((NP // 16,), jnp.float32),
    pltpu.VMEM_SHARED((NP,), jnp.float32),
]


@functools.partial(
    pl.kernel, mesh=_MESH,
    out_type=jax.ShapeDtypeStruct((2 * NP,), jnp.float32),
    scratch_types=_SCATTER_SCRATCH,
)
def _stage_a(sp_hbm, t_hbm, w_hbm, out_hbm, idx_v, w_v, sp_v, zb_v, acc_sh):
    def load_spikes(wid):
        pltpu.sync_copy(sp_hbm.at[wid], sp_v)

    _scatter_stage(t_hbm, w_hbm, out_hbm, idx_v, w_v, sp_v, zb_v, acc_sh,
                   load_spikes)


@functools.partial(
    pl.kernel, mesh=_MESH,
    out_type=jax.ShapeDtypeStruct((2 * NP,), jnp.float32),
    scratch_types=_SCATTER_SCRATCH + [
        pltpu.VMEM((RPT,), jnp.float32),
        pltpu.VMEM((RPT,), jnp.float32),
    ],
)
def _stage_b(p_hbm, t_hbm, w_hbm, out_hbm, idx_v, w_v, sp_v, zb_v, acc_sh,
             p0_v, p1_v):
    def load_spikes(wid):
        _threshold_spikes(p_hbm, wid * RPT, p0_v, p1_v, sp_v)

    _scatter_stage(t_hbm, w_hbm, out_hbm, idx_v, w_v, sp_v, zb_v, acc_sh,
                   load_spikes)


@functools.partial(
    pl.kernel, mesh=_MESH,
    out_type=jax.ShapeDtypeStruct((2 * NOUT,), jnp.float32),
    scratch_types=[
        pltpu.VMEM((RPT // 16, 16), jnp.int32),
        pltpu.VMEM((RPT // 16, 16), jnp.float32),
        pltpu.VMEM((RPT,), jnp.float32),
        pltpu.VMEM((NOUT,), jnp.float32),
        pltpu.VMEM_SHARED((NOUT,), jnp.float32),
        pltpu.VMEM((RPT,), jnp.float32),
        pltpu.VMEM((RPT,), jnp.float32),
    ],
)
def _stage_c(p_hbm, t_hbm, w_hbm, out_hbm, idx_v, w_v, sp_v, zb_v, acc_sh,
             p0_v, p1_v):
    cid = lax.axis_index("c")
    sid = lax.axis_index("s")
    wid = cid * 16 + sid

    @pl.when(sid == 0)
    def _():
        zeros = jnp.zeros((16,), jnp.float32)
        for i in range(NOUT // 16):
            zb_v[pl.ds(i * 16, 16)] = zeros
        pltpu.sync_copy(zb_v, acc_sh)

    pltpu.sync_copy(t_hbm.at[wid], idx_v)
    pltpu.sync_copy(w_hbm.at[wid], w_v)
    _threshold_spikes(p_hbm, wid * RPT, p0_v, p1_v, sp_v)
    plsc.subcore_barrier()
    for r in range(RPT // 16):
        w_v[r, pl.ds(0, 16)] = w_v[r, pl.ds(0, 16)] * sp_v[pl.ds(r * 16, 16)]

    def sc_body(r, carry):
        pltpu.sync_copy(w_v.at[r], acc_sh.at[idx_v.at[r]], add=True)
        return carry

    lax.fori_loop(0, RPT // 16, sc_body, 0)
    plsc.subcore_barrier()

    @pl.when(sid == 0)
    def _():
        pltpu.sync_copy(acc_sh, out_hbm.at[pl.ds(cid * NOUT, NOUT)])


@functools.partial(
    pl.kernel, mesh=_MESH,
    out_type=[jax.ShapeDtypeStruct((NOUT,), jnp.int32),
              jax.ShapeDtypeStruct((NOUT,), jnp.float32)],
    scratch_types=[
        pltpu.VMEM((NOUT,), jnp.float32),
        pltpu.VMEM((NOUT,), jnp.float32),
        pltpu.VMEM((16,), jnp.float32),
        pltpu.VMEM((NOUT,), jnp.int32),
        pltpu.VMEM((NOUT,), jnp.float32),
    ],
)
def _stage_d(p_hbm, m_hbm, ot_hbm, pv_hbm, p0_v, p1_v, m_v, ot_v, pv_v):
    wid = _wid()

    @pl.when(wid == 0)
    def _():
        pltpu.sync_copy(p_hbm.at[pl.ds(0, NOUT)], p0_v)
        pltpu.sync_copy(p_hbm.at[pl.ds(NOUT, NOUT)], p1_v)
        pltpu.sync_copy(m_hbm, m_v)
        mv = m_v[pl.ds(0, 16)]
        valid = mv >= 3.0
        factor = jnp.where(valid, jnp.exp(-(mv - 2.0) / TAU), 0.0)
        for g in range(NOUT // 16):
            s = pl.ds(g * 16, 16)
            acc = p0_v[s] + p1_v[s]
            fire = valid & ((acc * DECAY) >= THR)
            ot_v[s] = jnp.where(fire, 2, -1).astype(jnp.int32)
            pv_v[s] = acc * factor
        pltpu.sync_copy(ot_v, ot_hbm)
        pltpu.sync_copy(pv_v, pv_hbm)


def _pad_tw(T, W, n_targets):
    fan = T.shape[1]
    pad = NP - N
    pad_idx = (jax.lax.iota(jnp.int32, pad * fan) % n_targets).reshape(pad, fan)
    Tp = jnp.concatenate([T.astype(jnp.int32), pad_idx], axis=0)
    Wp = jnp.concatenate([W, jnp.zeros((pad, fan), jnp.float32)], axis=0)
    return Tp, Wp


def _chunked(Ap):
    return Ap.reshape(NW, RPT, FAN).transpose(0, 2, 1).reshape(NW, NROW, 128)


def kernel(input_spikes, max_timesteps, T0, W0, T1, W1, T2, W2):
    T0p, W0p = _pad_tw(T0, W0, N)
    T1p, W1p = _pad_tw(T1, W1, N)
    T2p, W2p = _pad_tw(T2, W2, NOUT)
    sp2 = jnp.concatenate([input_spikes.astype(jnp.float32) * 2.0,
                           jnp.zeros((NP - N,), jnp.float32)]).reshape(NW, RPT)
    t2 = T2p.reshape(NW, RPT // 16, 16)
    w2 = W2p.reshape(NW, RPT // 16, 16)
    mvec = jnp.full((16,), max_timesteps, jnp.float32)

    partials1 = _stage_a(sp2, _chunked(T0p), _chunked(W0p))
    partials2 = _stage_b(partials1, _chunked(T1p), _chunked(W1p))
    partials3 = _stage_c(partials2, t2, w2)
    out_times, pots = _stage_d(partials3, mvec)
    return out_times, pots


# speedup vs baseline: 129.6906x; 1.2114x over previous
import functools
import math

import numpy as np

import jax
import jax.numpy as jnp
from jax import lax
from jax.experimental import pallas as pl
from jax.experimental.pallas import tpu as pltpu
from jax.experimental.pallas import tpu_sc as plsc

N = 10000
NOUT = 128
FAN = 64
NP = 10240
NW = 32
RPT = NP // NW
NROW = RPT * FAN // 128
TAU = 20.0
THR = 1.0
DECAY = float(np.float32(math.exp(-1.0 / 20.0)))

_MESH = plsc.VectorSubcoreMesh(core_axis_name="c", subcore_axis_name="s")


def _wid():
    return lax.axis_index("c") * 16 + lax.axis_index("s")


def _zero_slice(zb_v, acc_sh, sid, span):
    zeros = jnp.zeros((16,), jnp.float32)
    for i in range(zb_v.shape[0] // 16):
        zb_v[pl.ds(i * 16, 16)] = zeros
    pltpu.sync_copy(zb_v.at[pl.ds(0, span)], acc_sh.at[pl.ds(sid * span, span)])


def _threshold_spikes(p_hbm, base, p0_v, p1_v, sp_v):
    pltpu.sync_copy(p_hbm.at[pl.ds(base, RPT)], p0_v)
    pltpu.sync_copy(p_hbm.at[pl.ds(NP + base, RPT)], p1_v)
    for i in range(RPT // 16):
        s = pl.ds(i * 16, 16)
        p = (p0_v[s] + p1_v[s]) * DECAY
        sp_v[s] = jnp.where(p >= THR, 1.0, 0.0).astype(jnp.float32)


def _scatter_stage(t_hbm, w_hbm, out_hbm, idx_v, w_v, sp_v, zb_v, acc_sh, sem,
                   load_spikes):
    cid = lax.axis_index("c")
    sid = lax.axis_index("s")
    wid = cid * 16 + sid
    h_idx = pltpu.async_copy(t_hbm.at[wid], idx_v, sem)
    h_w = pltpu.async_copy(w_hbm.at[wid], w_v, sem)
    _zero_slice(zb_v, acc_sh, sid, NP // 16)
    load_spikes(wid)
    h_idx.wait()
    h_w.wait()
    plsc.subcore_barrier()

    def mul_body(r, carry):
        for c8 in range(8):
            k = r * 128 + c8 * 16
            off = pl.multiple_of(lax.rem(k, RPT), 16)
            s = pl.ds(k, 16)
            w_v[s] = w_v[s] * sp_v[pl.ds(off, 16)]
        return carry

    lax.fori_loop(0, NROW, mul_body, 0)

    pltpu.sync_copy(w_v, acc_sh.at[idx_v], add=True)
    plsc.subcore_barrier()
    span = NP // 16
    pltpu.sync_copy(acc_sh.at[pl.ds(sid * span, span)],
                    out_hbm.at[pl.ds(cid * NP + sid * span, span)])


_SCATTER_SCRATCH = [
    pltpu.VMEM((NROW * 128,), jnp.int32),
    pltpu.VMEM((NROW * 128,), jnp.float32),
    pltpu.VMEM((RPT,), jnp.float32),
    pltpu.VMEM((NP // 16,), jnp.float32),
    pltpu.VMEM_SHARED((NP,), jnp.float32),
    pltpu.SemaphoreType.DMA,
]


@functools.partial(
    pl.kernel, mesh=_MESH,
    out_type=jax.ShapeDtypeStruct((2 * NP,), jnp.float32),
    scratch_types=_SCATTER_SCRATCH,
)
def _stage_a(sp_hbm, t_hbm, w_hbm, out_hbm, idx_v, w_v, sp_v, zb_v, acc_sh,
             sem):
    def load_spikes(wid):
        pltpu.sync_copy(sp_hbm.at[wid], sp_v)

    _scatter_stage(t_hbm, w_hbm, out_hbm, idx_v, w_v, sp_v, zb_v, acc_sh, sem,
                   load_spikes)


@functools.partial(
    pl.kernel, mesh=_MESH,
    out_type=jax.ShapeDtypeStruct((2 * NP,), jnp.float32),
    scratch_types=_SCATTER_SCRATCH + [
        pltpu.VMEM((RPT,), jnp.float32),
        pltpu.VMEM((RPT,), jnp.float32),
    ],
)
def _stage_b(p_hbm, t_hbm, w_hbm, out_hbm, idx_v, w_v, sp_v, zb_v, acc_sh,
             sem, p0_v, p1_v):
    def load_spikes(wid):
        _threshold_spikes(p_hbm, wid * RPT, p0_v, p1_v, sp_v)

    _scatter_stage(t_hbm, w_hbm, out_hbm, idx_v, w_v, sp_v, zb_v, acc_sh, sem,
                   load_spikes)


@functools.partial(
    pl.kernel, mesh=_MESH,
    out_type=jax.ShapeDtypeStruct((2 * NOUT,), jnp.float32),
    scratch_types=[
        pltpu.VMEM((RPT,), jnp.int32),
        pltpu.VMEM((RPT,), jnp.float32),
        pltpu.VMEM((RPT,), jnp.float32),
        pltpu.VMEM((NOUT,), jnp.float32),
        pltpu.VMEM_SHARED((NOUT,), jnp.float32),
        pltpu.VMEM((RPT,), jnp.float32),
        pltpu.VMEM((RPT,), jnp.float32),
    ],
)
def _stage_c(p_hbm, t_hbm, w_hbm, out_hbm, idx_v, w_v, sp_v, zb_v, acc_sh,
             p0_v, p1_v):
    cid = lax.axis_index("c")
    sid = lax.axis_index("s")
    wid = cid * 16 + sid

    @pl.when(sid == 0)
    def _():
        zeros = jnp.zeros((16,), jnp.float32)
        for i in range(NOUT // 16):
            zb_v[pl.ds(i * 16, 16)] = zeros
        pltpu.sync_copy(zb_v, acc_sh)

    pltpu.sync_copy(t_hbm.at[wid], idx_v)
    pltpu.sync_copy(w_hbm.at[wid], w_v)
    _threshold_spikes(p_hbm, wid * RPT, p0_v, p1_v, sp_v)
    plsc.subcore_barrier()
    for r in range(RPT // 16):
        s = pl.ds(r * 16, 16)
        w_v[s] = w_v[s] * sp_v[s]

    pltpu.sync_copy(w_v, acc_sh.at[idx_v], add=True)
    plsc.subcore_barrier()

    @pl.when(sid == 0)
    def _():
        pltpu.sync_copy(acc_sh, out_hbm.at[pl.ds(cid * NOUT, NOUT)])


@functools.partial(
    pl.kernel, mesh=_MESH,
    out_type=[jax.ShapeDtypeStruct((NOUT,), jnp.int32),
              jax.ShapeDtypeStruct((NOUT,), jnp.float32)],
    scratch_types=[
        pltpu.VMEM((NOUT,), jnp.float32),
        pltpu.VMEM((NOUT,), jnp.float32),
        pltpu.VMEM((16,), jnp.float32),
        pltpu.VMEM((NOUT,), jnp.int32),
        pltpu.VMEM((NOUT,), jnp.float32),
    ],
)
def _stage_d(p_hbm, m_hbm, ot_hbm, pv_hbm, p0_v, p1_v, m_v, ot_v, pv_v):
    wid = _wid()

    @pl.when(wid == 0)
    def _():
        pltpu.sync_copy(p_hbm.at[pl.ds(0, NOUT)], p0_v)
        pltpu.sync_copy(p_hbm.at[pl.ds(NOUT, NOUT)], p1_v)
        pltpu.sync_copy(m_hbm, m_v)
        mv = m_v[pl.ds(0, 16)]
        valid = mv >= 3.0
        factor = jnp.where(valid, jnp.exp(-(mv - 2.0) / TAU), 0.0)
        for g in range(NOUT // 16):
            s = pl.ds(g * 16, 16)
            acc = p0_v[s] + p1_v[s]
            fire = valid & ((acc * DECAY) >= THR)
            ot_v[s] = jnp.where(fire, 2, -1).astype(jnp.int32)
            pv_v[s] = acc * factor
        pltpu.sync_copy(ot_v, ot_hbm)
        pltpu.sync_copy(pv_v, pv_hbm)


def _pad_tw(T, W, n_targets):
    fan = T.shape[1]
    pad = NP - N
    pad_idx = (jax.lax.iota(jnp.int32, pad * fan) % n_targets).reshape(pad, fan)
    Tp = jnp.concatenate([T.astype(jnp.int32), pad_idx], axis=0)
    Wp = jnp.concatenate([W, jnp.zeros((pad, fan), jnp.float32)], axis=0)
    return Tp, Wp


def _chunked(Ap):
    return Ap.reshape(NW, RPT, FAN).transpose(0, 2, 1).reshape(NW, RPT * FAN)


def kernel(input_spikes, max_timesteps, T0, W0, T1, W1, T2, W2):
    T0p, W0p = _pad_tw(T0, W0, N)
    T1p, W1p = _pad_tw(T1, W1, N)
    T2p, W2p = _pad_tw(T2, W2, NOUT)
    sp2 = jnp.concatenate([input_spikes.astype(jnp.float32) * 2.0,
                           jnp.zeros((NP - N,), jnp.float32)]).reshape(NW, RPT)
    t2 = T2p.reshape(NW, RPT)
    w2 = W2p.reshape(NW, RPT)
    mvec = jnp.full((16,), max_timesteps, jnp.float32)

    partials1 = _stage_a(sp2, _chunked(T0p), _chunked(W0p))
    partials2 = _stage_b(partials1, _chunked(T1p), _chunked(W1p))
    partials3 = _stage_c(partials2, t2, w2)
    out_times, pots = _stage_d(partials3, mvec)
    return out_times, pots
